# Initial kernel scaffold; baseline (speedup 1.0000x reference)
import jax
import jax.numpy as jnp
from jax.experimental import pallas as pl


def kernel(inputs, table):
    raise NotImplementedError("write your pallas kernel here")


# speedup vs baseline: 1.0902x; 1.0902x over previous
import functools

import jax
import jax.numpy as jnp
from jax import lax
from jax.experimental import pallas as pl
from jax.experimental.pallas import tpu as pltpu
from jax.experimental.pallas import tpu_sc as plsc

NC = 2
NS = 16
NW = NC * NS

IDX_W = 128
CHUNK = 512
SUB = CHUNK // IDX_W


def _make_gather(B, V, D):
    assert B % (NW * CHUNK) == 0
    bpw = B // NW
    n_chunks = bpw // CHUNK
    idx_rows = bpw // IDX_W

    mesh = plsc.VectorSubcoreMesh(
        core_axis_name="c", subcore_axis_name="s",
        num_cores=NC, num_subcores=NS)

    @functools.partial(
        pl.kernel,
        out_type=jax.ShapeDtypeStruct((B, D), jnp.float32),
        mesh=mesh,
        scratch_types=[
            pltpu.VMEM((idx_rows, IDX_W), jnp.int32),
            pltpu.VMEM((CHUNK, D), jnp.float32),
            pltpu.SemaphoreType.DMA,
        ],
        compiler_params=pltpu.CompilerParams(use_tc_tiling_on_sc=False),
    )
    def gather_kernel(idx_hbm, table_hbm, out_hbm, idx_v, buf, gsem):
        wid = lax.axis_index("s") * NC + lax.axis_index("c")
        row_base = wid * bpw
        idx_base = wid * idx_rows

        pltpu.sync_copy(idx_hbm.at[pl.ds(idx_base, idx_rows)], idx_v)

        def body(g, carry):
            copies = []
            for c in range(SUB):
                copies.append(pltpu.async_copy(
                    table_hbm.at[idx_v.at[g * SUB + c]],
                    buf.at[pl.ds(c * IDX_W, IDX_W), :],
                    gsem))
            for cp in copies:
                cp.wait()
            pltpu.sync_copy(buf, out_hbm.at[pl.ds(row_base + g * CHUNK, CHUNK)])
            return carry

        lax.fori_loop(0, n_chunks, body, 0)

    return gather_kernel


def kernel(inputs, table):
    n, h = inputs.shape
    V, D = table.shape
    B = n * h
    idx = inputs.reshape(B // IDX_W, IDX_W).astype(jnp.int32)
    out = _make_gather(B, V, D)(idx, table)
    return out.reshape(n, h, D)


# speedup vs baseline: 1.1132x; 1.0211x over previous
import functools

import jax
import jax.numpy as jnp
from jax import lax
from jax.experimental import pallas as pl
from jax.experimental.pallas import tpu as pltpu
from jax.experimental.pallas import tpu_sc as plsc

NC = 2
NS = 16
NW = NC * NS

IDX_W = 128
CHUNK = 256
SUB = CHUNK // IDX_W
NBUF = 10


def _make_gather(B, V, D):
    assert B % (NW * CHUNK) == 0
    bpw = B // NW
    n_chunks = bpw // CHUNK
    idx_rows = bpw // IDX_W
    assert n_chunks > NBUF

    mesh = plsc.VectorSubcoreMesh(
        core_axis_name="c", subcore_axis_name="s",
        num_cores=NC, num_subcores=NS)

    @functools.partial(
        pl.kernel,
        out_type=jax.ShapeDtypeStruct((B, D), jnp.float32),
        mesh=mesh,
        scratch_types=[
            pltpu.VMEM((idx_rows, IDX_W), jnp.int32),
            pltpu.VMEM((NBUF * CHUNK, D), jnp.float32),
            pltpu.SemaphoreType.DMA((NBUF,)),
            pltpu.SemaphoreType.DMA((NBUF,)),
        ],
        compiler_params=pltpu.CompilerParams(use_tc_tiling_on_sc=False),
    )
    def gather_kernel(idx_hbm, table_hbm, out_hbm, idx_v, buf, gsem, ssem):
        wid = lax.axis_index("s") * NC + lax.axis_index("c")
        row_base = wid * bpw
        idx_base = wid * idx_rows

        pltpu.sync_copy(idx_hbm.at[pl.ds(idx_base, idx_rows)], idx_v)

        def fire_gathers(j, b):
            for c in range(SUB):
                pltpu.async_copy(
                    table_hbm.at[idx_v.at[j * SUB + c]],
                    buf.at[pl.ds(b * CHUNK + c * IDX_W, IDX_W), :],
                    gsem.at[b])

        def drain_gathers(b):
            for c in range(SUB):
                pltpu.make_async_copy(
                    table_hbm.at[idx_v.at[0]],
                    buf.at[pl.ds(b * CHUNK + c * IDX_W, IDX_W), :],
                    gsem.at[b]).wait()

        def fire_store(j, b):
            pltpu.async_copy(
                buf.at[pl.ds(b * CHUNK, CHUNK), :],
                out_hbm.at[pl.ds(row_base + j * CHUNK, CHUNK)],
                ssem.at[b])

        def wait_store(b):
            pltpu.make_async_copy(
                buf.at[pl.ds(b * CHUNK, CHUNK), :],
                out_hbm.at[pl.ds(row_base, CHUNK)],
                ssem.at[b]).wait()

        for j in range(NBUF):
            fire_gathers(j, j)

        def body(j, carry):
            b = j % NBUF
            drain_gathers(b)
            fire_store(j, b)
            jp = j - 1 + NBUF

            @pl.when(jnp.logical_and(j >= 1, jp < n_chunks))
            def _():
                bp = (j - 1) % NBUF
                wait_store(bp)
                fire_gathers(jp, bp)

            return carry

        lax.fori_loop(0, n_chunks, body, 0)

        for j in range(n_chunks - NBUF, n_chunks):
            wait_store(j % NBUF)

    return gather_kernel


def kernel(inputs, table):
    n, h = inputs.shape
    V, D = table.shape
    B = n * h
    idx = inputs.reshape(B // IDX_W, IDX_W).astype(jnp.int32)
    out = _make_gather(B, V, D)(idx, table)
    return out.reshape(n, h, D)
